# Initial kernel scaffold; baseline (speedup 1.0000x reference)
import jax
import jax.numpy as jnp
from jax.experimental import pallas as pl


def kernel(gt_boxes, bbox_regression, anchors):
    raise NotImplementedError("write your pallas kernel here")


# speedup vs baseline: 13.2616x; 13.2616x over previous
import functools

import jax
import jax.numpy as jnp
from jax import lax
from jax.experimental import pallas as pl
from jax.experimental.pallas import tpu as pltpu

_B, _N, _G = 8, 20000, 100
_FG_T, _BG_T = 0.5, 0.4
_GP = 128
_NPAD = 20480
_CH = 2560
_NCH = _NPAD // _CH


def _body(aT_ref, colpen_ref, gtc_ref, gt8_ref, regT_ref, out_ref, iou_ref):
    b = pl.program_id(0)

    gx1 = gtc_ref[0, :, 0:1]
    gy1 = gtc_ref[0, :, 1:2]
    gx2 = gtc_ref[0, :, 2:3]
    gy2 = gtc_ref[0, :, 3:4]
    garea = gtc_ref[0, :, 4:5]
    rowpen = gtc_ref[0, :, 5:6]
    rowpen2 = rowpen * 2.0

    def iou_chunk(c, hpg):
        sl = pl.ds(c * _CH, _CH)
        ax1 = aT_ref[0:1, sl]
        ay1 = aT_ref[1:2, sl]
        ax2 = aT_ref[2:3, sl]
        ay2 = aT_ref[3:4, sl]
        aarea = (ax2 - ax1) * (ay2 - ay1)
        w = jnp.maximum(jnp.minimum(gx2, ax2) - jnp.maximum(gx1, ax1), 0.0)
        h = jnp.maximum(jnp.minimum(gy2, ay2) - jnp.maximum(gy1, ay1), 0.0)
        inter = w * h
        iou = inter / (garea + aarea - inter)
        ioum = iou - rowpen2 - colpen_ref[0:1, sl]
        iou_ref[:, sl] = ioum
        return jnp.maximum(hpg, jnp.max(ioum, axis=1, keepdims=True))

    hpg = lax.fori_loop(
        0, _NCH, iou_chunk, jnp.full((_GP, 1), -1e9, jnp.float32))
    hpg = hpg - rowpen * 100.0

    giota = lax.broadcasted_iota(jnp.int32, (_GP, 1), 0).astype(jnp.float32)

    def loss_chunk(c, carry):
        ls, nf = carry
        sl = pl.ds(c * _CH, _CH)
        iou = iou_ref[:, sl]
        mv = jnp.max(iou, axis=0, keepdims=True)
        idxv = jnp.min(jnp.where(iou == mv, giota, 1000.0),
                       axis=0, keepdims=True)
        upd = jnp.max(jnp.where(iou == hpg, 1.0, 0.0),
                      axis=0, keepdims=True)
        fg = jnp.maximum(jnp.where(mv >= _FG_T, 1.0, 0.0), upd)
        sel = jnp.where(giota == idxv, 1.0, 0.0)
        mgt = jnp.dot(gt8_ref[0], sel,
                      preferred_element_type=jnp.float32)

        ax1 = aT_ref[0:1, sl]
        ay1 = aT_ref[1:2, sl]
        ax2 = aT_ref[2:3, sl]
        ay2 = aT_ref[3:4, sl]
        ew = ax2 - ax1
        eh = ay2 - ay1
        ecx = ax1 + 0.5 * ew
        ecy = ay1 + 0.5 * eh
        gw = mgt[2:3] - mgt[0:1]
        gh = mgt[3:4] - mgt[1:2]
        gcx = mgt[0:1] + 0.5 * gw
        gcy = mgt[1:2] + 0.5 * gh
        tx = (gcx - ecx) / ew
        ty = (gcy - ecy) / eh
        tw = jnp.log(gw / ew)
        th = jnp.log(gh / eh)
        d = (jnp.abs(regT_ref[0, 0:1, sl] - tx)
             + jnp.abs(regT_ref[0, 1:2, sl] - ty)
             + jnp.abs(regT_ref[0, 2:3, sl] - tw)
             + jnp.abs(regT_ref[0, 3:4, sl] - th))
        ls = ls + jnp.sum(d * fg)
        nf = nf + jnp.sum(fg)
        return ls, nf

    ls, nf = lax.fori_loop(
        0, _NCH, loss_chunk,
        (jnp.zeros((), jnp.float32), jnp.zeros((), jnp.float32)))
    li = ls / jnp.maximum(1.0, nf) * (1.0 / _B)

    li2 = jnp.full((1, 1), li, jnp.float32)

    @pl.when(b == 0)
    def _():
        out_ref[:, :] = li2

    @pl.when(b != 0)
    def _():
        out_ref[:, :] = out_ref[:, :] + li2


@jax.jit
def kernel(gt_boxes, bbox_regression, anchors):
    f32 = jnp.float32
    pad_a = jnp.tile(jnp.array([[0.0], [0.0], [1.0], [1.0]], f32),
                     (1, _NPAD - _N))
    aT = jnp.concatenate([anchors.T.astype(f32), pad_a], axis=1)
    colpen = jnp.concatenate(
        [jnp.zeros((1, _N), f32), jnp.full((1, _NPAD - _N), 2.0, f32)],
        axis=1)
    pad_g = jnp.tile(
        jnp.array([[-1000.0, -1000.0, -999.0, -999.0]], f32),
        (_B, _GP - _G, 1))
    gtp = jnp.concatenate([gt_boxes.astype(f32), pad_g], axis=1)
    area = ((gtp[:, :, 2] - gtp[:, :, 0])
            * (gtp[:, :, 3] - gtp[:, :, 1]))[:, :, None]
    rowpen = jnp.concatenate(
        [jnp.zeros((_B, _G, 1), f32), jnp.ones((_B, _GP - _G, 1), f32)],
        axis=1)
    gtc = jnp.concatenate(
        [gtp, area, rowpen, jnp.zeros((_B, _GP, 2), f32)], axis=2)
    gt8 = jnp.concatenate(
        [jnp.where(rowpen.transpose(0, 2, 1) > 0, 0.0,
                   gtp.transpose(0, 2, 1)),
         jnp.zeros((_B, 4, _GP), f32)], axis=1)
    regT = jnp.concatenate(
        [bbox_regression.astype(f32).transpose(0, 2, 1),
         jnp.zeros((_B, 4, _NPAD - _N), f32)], axis=2)

    out = pl.pallas_call(
        _body,
        grid=(_B,),
        in_specs=[
            pl.BlockSpec((4, _NPAD), lambda b: (0, 0)),
            pl.BlockSpec((1, _NPAD), lambda b: (0, 0)),
            pl.BlockSpec((1, _GP, 8), lambda b: (b, 0, 0)),
            pl.BlockSpec((1, 8, _GP), lambda b: (b, 0, 0)),
            pl.BlockSpec((1, 4, _NPAD), lambda b: (b, 0, 0)),
        ],
        out_specs=pl.BlockSpec((1, 1), lambda b: (0, 0)),
        out_shape=jax.ShapeDtypeStruct((1, 1), f32),
        scratch_shapes=[pltpu.VMEM((_GP, _NPAD), f32)],
    )(aT, colpen, gtc, gt8, regT)
    return out[0, 0]


# speedup vs baseline: 15.9441x; 1.2023x over previous
import functools

import jax
import jax.numpy as jnp
from jax import lax
from jax.experimental import pallas as pl
from jax.experimental.pallas import tpu as pltpu

_B, _N, _G = 8, 20000, 100
_FG_T, _BG_T = 0.5, 0.4
_GP = 104
_NPAD = 20480
_CH = 2560
_NCH = _NPAD // _CH


def _body(aT_ref, colval_ref, gtc_ref, gt8_ref, regT_ref, out_ref, iou_ref):
    b = pl.program_id(0)

    gx1 = gtc_ref[0, :, 0:1]
    gy1 = gtc_ref[0, :, 1:2]
    gx2 = gtc_ref[0, :, 2:3]
    gy2 = gtc_ref[0, :, 3:4]
    garea = gtc_ref[0, :, 4:5]
    rowpen = gtc_ref[0, :, 5:6]

    def iou_chunk(c, hpg):
        sl = pl.ds(c * _CH, _CH)
        ax1 = aT_ref[0:1, sl]
        ay1 = aT_ref[1:2, sl]
        ax2 = aT_ref[2:3, sl]
        ay2 = aT_ref[3:4, sl]
        aarea = aT_ref[4:5, sl]
        w = jnp.maximum(jnp.minimum(gx2, ax2) - jnp.maximum(gx1, ax1), 0.0)
        h = jnp.maximum(jnp.minimum(gy2, ay2) - jnp.maximum(gy1, ay1), 0.0)
        inter = w * h
        iou = inter / (garea + aarea - inter)
        iou_ref[:, sl] = iou
        return jnp.maximum(hpg, jnp.max(iou, axis=1, keepdims=True))

    hpg = lax.fori_loop(
        0, _NCH, iou_chunk, jnp.full((_GP, 1), -1e9, jnp.float32))
    hpg = hpg - rowpen * 100.0

    giota = lax.broadcasted_iota(jnp.int32, (_GP, 1), 0).astype(jnp.float32)

    def loss_chunk(c, carry):
        ls, nf = carry
        sl = pl.ds(c * _CH, _CH)
        iou = iou_ref[:, sl]
        mv = jnp.max(iou, axis=0, keepdims=True)
        idxv = jnp.min(jnp.where(iou == mv, giota, 1000.0),
                       axis=0, keepdims=True)
        upd = jnp.max(jnp.where(iou == hpg, 1.0, 0.0),
                      axis=0, keepdims=True)
        fg = (jnp.maximum(jnp.where(mv >= _FG_T, 1.0, 0.0), upd)
              * colval_ref[0:1, sl])
        sel = jnp.where(giota == idxv, 1.0, 0.0)
        mgt = jnp.dot(gt8_ref[0], sel,
                      preferred_element_type=jnp.float32)

        ax1 = aT_ref[0:1, sl]
        ay1 = aT_ref[1:2, sl]
        ax2 = aT_ref[2:3, sl]
        ay2 = aT_ref[3:4, sl]
        ew = ax2 - ax1
        eh = ay2 - ay1
        ecx = ax1 + 0.5 * ew
        ecy = ay1 + 0.5 * eh
        gw = mgt[2:3] - mgt[0:1]
        gh = mgt[3:4] - mgt[1:2]
        gcx = mgt[0:1] + 0.5 * gw
        gcy = mgt[1:2] + 0.5 * gh
        tx = (gcx - ecx) / ew
        ty = (gcy - ecy) / eh
        tw = jnp.log(gw / ew)
        th = jnp.log(gh / eh)
        d = (jnp.abs(regT_ref[0, 0:1, sl] - tx)
             + jnp.abs(regT_ref[0, 1:2, sl] - ty)
             + jnp.abs(regT_ref[0, 2:3, sl] - tw)
             + jnp.abs(regT_ref[0, 3:4, sl] - th))
        ls = ls + jnp.sum(d * fg)
        nf = nf + jnp.sum(fg)
        return ls, nf

    ls, nf = lax.fori_loop(
        0, _NCH, loss_chunk,
        (jnp.zeros((), jnp.float32), jnp.zeros((), jnp.float32)))
    li = ls / jnp.maximum(1.0, nf) * (1.0 / _B)
    li2 = jnp.full((1, 1), li, jnp.float32)

    @pl.when(b == 0)
    def _():
        out_ref[:, :] = li2

    @pl.when(b != 0)
    def _():
        out_ref[:, :] = out_ref[:, :] + li2


@jax.jit
def kernel(gt_boxes, bbox_regression, anchors):
    f32 = jnp.float32
    pad_a = jnp.tile(jnp.array([[4000.0], [4000.0], [4001.0], [4001.0]], f32),
                     (1, _NPAD - _N))
    a4 = jnp.concatenate([anchors.T.astype(f32), pad_a], axis=1)
    aarea = ((a4[2:3] - a4[0:1]) * (a4[3:4] - a4[1:2]))
    aT = jnp.concatenate([a4, aarea], axis=0)
    colval = jnp.concatenate(
        [jnp.ones((1, _N), f32), jnp.zeros((1, _NPAD - _N), f32)], axis=1)
    pad_g = jnp.tile(
        jnp.array([[-4000.0, -4000.0, -3999.0, -3999.0]], f32),
        (_B, _GP - _G, 1))
    gtp = jnp.concatenate([gt_boxes.astype(f32), pad_g], axis=1)
    area = ((gtp[:, :, 2] - gtp[:, :, 0])
            * (gtp[:, :, 3] - gtp[:, :, 1]))[:, :, None]
    rowpen = jnp.concatenate(
        [jnp.zeros((_B, _G, 1), f32), jnp.ones((_B, _GP - _G, 1), f32)],
        axis=1)
    gtc = jnp.concatenate(
        [gtp, area, rowpen, jnp.zeros((_B, _GP, 2), f32)], axis=2)
    gt8 = jnp.concatenate(
        [jnp.where(rowpen.transpose(0, 2, 1) > 0, 0.0,
                   gtp.transpose(0, 2, 1)),
         jnp.zeros((_B, 4, _GP), f32)], axis=1)
    regT = jnp.concatenate(
        [bbox_regression.astype(f32).transpose(0, 2, 1),
         jnp.zeros((_B, 4, _NPAD - _N), f32)], axis=2)

    out = pl.pallas_call(
        _body,
        grid=(_B,),
        in_specs=[
            pl.BlockSpec((5, _NPAD), lambda b: (0, 0)),
            pl.BlockSpec((1, _NPAD), lambda b: (0, 0)),
            pl.BlockSpec((1, _GP, 8), lambda b: (b, 0, 0)),
            pl.BlockSpec((1, 8, _GP), lambda b: (b, 0, 0)),
            pl.BlockSpec((1, 4, _NPAD), lambda b: (b, 0, 0)),
        ],
        out_specs=pl.BlockSpec((1, 1), lambda b: (0, 0)),
        out_shape=jax.ShapeDtypeStruct((1, 1), f32),
        scratch_shapes=[pltpu.VMEM((_GP, _NPAD), f32)],
    )(aT, colval, gtc, gt8, regT)
    return out[0, 0]


# speedup vs baseline: 17.2491x; 1.0818x over previous
import functools

import jax
import jax.numpy as jnp
from jax import lax
from jax.experimental import pallas as pl
from jax.experimental.pallas import tpu as pltpu

_B, _N, _G = 8, 20000, 100
_FG_T, _BG_T = 0.5, 0.4
_GP = 104
_NPAD = 20480
_CH = 2560
_NCH = _NPAD // _CH


def _body(aT_ref, colval_ref, gtc_ref, gt8_ref, regT_ref, out_ref, iou_ref):
    b = pl.program_id(0)

    gx1 = gtc_ref[0, :, 0:1]
    gy1 = gtc_ref[0, :, 1:2]
    gx2 = gtc_ref[0, :, 2:3]
    gy2 = gtc_ref[0, :, 3:4]
    garea = gtc_ref[0, :, 4:5]
    rowpen = gtc_ref[0, :, 5:6]

    def iou_chunk(c, hacc):
        sl = pl.ds(c * _CH, _CH)
        ax1 = aT_ref[0:1, sl]
        ay1 = aT_ref[1:2, sl]
        ax2 = aT_ref[2:3, sl]
        ay2 = aT_ref[3:4, sl]
        aarea = aT_ref[4:5, sl]
        w = jnp.maximum(jnp.minimum(gx2, ax2) - jnp.maximum(gx1, ax1), 0.0)
        h = jnp.maximum(jnp.minimum(gy2, ay2) - jnp.maximum(gy1, ay1), 0.0)
        inter = w * h
        iou = inter / (garea + aarea - inter)
        iou_ref[:, sl] = iou
        for k in range(_CH // 128):
            hacc = jnp.maximum(hacc, iou[:, k * 128:(k + 1) * 128])
        return hacc

    hacc = lax.fori_loop(
        0, _NCH, iou_chunk, jnp.full((_GP, 128), -1e9, jnp.float32))
    hpg = jnp.max(hacc, axis=1, keepdims=True)
    hpg = hpg + rowpen * 100.0

    giota = lax.broadcasted_iota(jnp.int32, (_GP, 1), 0).astype(jnp.float32)

    def loss_chunk(c, carry):
        ls, nf = carry
        sl = pl.ds(c * _CH, _CH)
        iou = iou_ref[:, sl]
        mv = jnp.max(iou, axis=0, keepdims=True)
        idxv = jnp.min(jnp.where(iou == mv, giota, 1000.0),
                       axis=0, keepdims=True)
        maxdiff = jnp.max(iou - hpg, axis=0, keepdims=True)
        upd = jnp.where(maxdiff == 0.0, 1.0, 0.0)
        fg = (jnp.maximum(jnp.where(mv >= _FG_T, 1.0, 0.0), upd)
              * colval_ref[0:1, sl])
        sel = jnp.where(giota == idxv, 1.0, 0.0)
        mgt = jnp.dot(gt8_ref[0], sel,
                      preferred_element_type=jnp.float32)

        ax1 = aT_ref[0:1, sl]
        ay1 = aT_ref[1:2, sl]
        ax2 = aT_ref[2:3, sl]
        ay2 = aT_ref[3:4, sl]
        ew = ax2 - ax1
        eh = ay2 - ay1
        ecx = ax1 + 0.5 * ew
        ecy = ay1 + 0.5 * eh
        gw = mgt[2:3] - mgt[0:1]
        gh = mgt[3:4] - mgt[1:2]
        gcx = mgt[0:1] + 0.5 * gw
        gcy = mgt[1:2] + 0.5 * gh
        tx = (gcx - ecx) / ew
        ty = (gcy - ecy) / eh
        tw = jnp.log(gw / ew)
        th = jnp.log(gh / eh)
        d = (jnp.abs(regT_ref[0, 0:1, sl] - tx)
             + jnp.abs(regT_ref[0, 1:2, sl] - ty)
             + jnp.abs(regT_ref[0, 2:3, sl] - tw)
             + jnp.abs(regT_ref[0, 3:4, sl] - th))
        ls = ls + jnp.sum(d * fg)
        nf = nf + jnp.sum(fg)
        return ls, nf

    ls, nf = lax.fori_loop(
        0, _NCH, loss_chunk,
        (jnp.zeros((), jnp.float32), jnp.zeros((), jnp.float32)))
    li = ls / jnp.maximum(1.0, nf) * (1.0 / _B)
    li2 = jnp.full((1, 1), li, jnp.float32)

    @pl.when(b == 0)
    def _():
        out_ref[:, :] = li2

    @pl.when(b != 0)
    def _():
        out_ref[:, :] = out_ref[:, :] + li2


@jax.jit
def kernel(gt_boxes, bbox_regression, anchors):
    f32 = jnp.float32
    pad_a = jnp.tile(jnp.array([[4000.0], [4000.0], [4001.0], [4001.0]], f32),
                     (1, _NPAD - _N))
    a4 = jnp.concatenate([anchors.T.astype(f32), pad_a], axis=1)
    aarea = ((a4[2:3] - a4[0:1]) * (a4[3:4] - a4[1:2]))
    aT = jnp.concatenate([a4, aarea], axis=0)
    colval = jnp.concatenate(
        [jnp.ones((1, _N), f32), jnp.zeros((1, _NPAD - _N), f32)], axis=1)
    pad_g = jnp.tile(
        jnp.array([[-4000.0, -4000.0, -3999.0, -3999.0]], f32),
        (_B, _GP - _G, 1))
    gtp = jnp.concatenate([gt_boxes.astype(f32), pad_g], axis=1)
    area = ((gtp[:, :, 2] - gtp[:, :, 0])
            * (gtp[:, :, 3] - gtp[:, :, 1]))[:, :, None]
    rowpen = jnp.concatenate(
        [jnp.zeros((_B, _G, 1), f32), jnp.ones((_B, _GP - _G, 1), f32)],
        axis=1)
    gtc = jnp.concatenate(
        [gtp, area, rowpen, jnp.zeros((_B, _GP, 2), f32)], axis=2)
    gt8 = jnp.concatenate(
        [jnp.where(rowpen.transpose(0, 2, 1) > 0, 0.0,
                   gtp.transpose(0, 2, 1)),
         jnp.zeros((_B, 4, _GP), f32)], axis=1)
    regT = jnp.concatenate(
        [bbox_regression.astype(f32).transpose(0, 2, 1),
         jnp.zeros((_B, 4, _NPAD - _N), f32)], axis=2)

    out = pl.pallas_call(
        _body,
        grid=(_B,),
        in_specs=[
            pl.BlockSpec((5, _NPAD), lambda b: (0, 0)),
            pl.BlockSpec((1, _NPAD), lambda b: (0, 0)),
            pl.BlockSpec((1, _GP, 8), lambda b: (b, 0, 0)),
            pl.BlockSpec((1, 8, _GP), lambda b: (b, 0, 0)),
            pl.BlockSpec((1, 4, _NPAD), lambda b: (b, 0, 0)),
        ],
        out_specs=pl.BlockSpec((1, 1), lambda b: (0, 0)),
        out_shape=jax.ShapeDtypeStruct((1, 1), f32),
        scratch_shapes=[pltpu.VMEM((_GP, _NPAD), f32)],
    )(aT, colval, gtc, gt8, regT)
    return out[0, 0]


# speedup vs baseline: 18.4305x; 1.0685x over previous
import functools

import jax
import jax.numpy as jnp
from jax import lax
from jax.experimental import pallas as pl
from jax.experimental.pallas import tpu as pltpu

_B, _N, _G = 8, 20000, 100
_FG_T, _BG_T = 0.5, 0.4
_GP = 104
_NPAD = 20480
_CH = 5120
_NCH = _NPAD // _CH


def _body(aT_ref, colval_ref, gtc_ref, gt8_ref, regT_ref, out_ref, iou_ref):
    b = pl.program_id(0)

    gx1 = gtc_ref[0, :, 0:1]
    gy1 = gtc_ref[0, :, 1:2]
    gx2 = gtc_ref[0, :, 2:3]
    gy2 = gtc_ref[0, :, 3:4]
    garea = gtc_ref[0, :, 4:5]
    rowpen = gtc_ref[0, :, 5:6]

    def iou_chunk(c, hacc):
        sl = pl.ds(c * _CH, _CH)
        ax1 = aT_ref[0:1, sl]
        ay1 = aT_ref[1:2, sl]
        ax2 = aT_ref[2:3, sl]
        ay2 = aT_ref[3:4, sl]
        aarea = aT_ref[4:5, sl]
        w = jnp.maximum(jnp.minimum(gx2, ax2) - jnp.maximum(gx1, ax1), 0.0)
        h = jnp.maximum(jnp.minimum(gy2, ay2) - jnp.maximum(gy1, ay1), 0.0)
        inter = w * h
        iou = inter / (garea + aarea - inter)
        iou_ref[:, sl] = iou
        for k in range(_CH // 128):
            hacc = jnp.maximum(hacc, iou[:, k * 128:(k + 1) * 128])
        return hacc

    hacc = lax.fori_loop(
        0, _NCH, iou_chunk, jnp.full((_GP, 128), -1e9, jnp.float32))
    hpg = jnp.max(hacc, axis=1, keepdims=True)
    hpg = hpg + rowpen * 100.0

    giota = lax.broadcasted_iota(jnp.int32, (_GP, 1), 0).astype(jnp.float32)

    def loss_chunk(c, carry):
        ls, nf = carry
        sl = pl.ds(c * _CH, _CH)
        iou = iou_ref[:, sl]
        mv = jnp.max(iou, axis=0, keepdims=True)
        idxv = jnp.min(jnp.where(iou == mv, giota, 1000.0),
                       axis=0, keepdims=True)
        maxdiff = jnp.max(iou - hpg, axis=0, keepdims=True)
        upd = jnp.where(maxdiff == 0.0, 1.0, 0.0)
        fg = (jnp.maximum(jnp.where(mv >= _FG_T, 1.0, 0.0), upd)
              * colval_ref[0:1, sl])
        sel = jnp.where(giota == idxv, 1.0, 0.0)
        mgt = jnp.dot(gt8_ref[0], sel,
                      preferred_element_type=jnp.float32)

        ax1 = aT_ref[0:1, sl]
        ay1 = aT_ref[1:2, sl]
        ax2 = aT_ref[2:3, sl]
        ay2 = aT_ref[3:4, sl]
        ew = ax2 - ax1
        eh = ay2 - ay1
        ecx = ax1 + 0.5 * ew
        ecy = ay1 + 0.5 * eh
        gw = mgt[2:3] - mgt[0:1]
        gh = mgt[3:4] - mgt[1:2]
        gcx = mgt[0:1] + 0.5 * gw
        gcy = mgt[1:2] + 0.5 * gh
        tx = (gcx - ecx) / ew
        ty = (gcy - ecy) / eh
        tw = jnp.log(gw / ew)
        th = jnp.log(gh / eh)
        d = (jnp.abs(regT_ref[0, 0:1, sl] - tx)
             + jnp.abs(regT_ref[0, 1:2, sl] - ty)
             + jnp.abs(regT_ref[0, 2:3, sl] - tw)
             + jnp.abs(regT_ref[0, 3:4, sl] - th))
        ls = ls + jnp.sum(d * fg)
        nf = nf + jnp.sum(fg)
        return ls, nf

    ls, nf = lax.fori_loop(
        0, _NCH, loss_chunk,
        (jnp.zeros((), jnp.float32), jnp.zeros((), jnp.float32)))
    li = ls / jnp.maximum(1.0, nf) * (1.0 / _B)
    li2 = jnp.full((1, 1), li, jnp.float32)

    @pl.when(b == 0)
    def _():
        out_ref[:, :] = li2

    @pl.when(b != 0)
    def _():
        out_ref[:, :] = out_ref[:, :] + li2


@jax.jit
def kernel(gt_boxes, bbox_regression, anchors):
    f32 = jnp.float32
    pad_a = jnp.tile(jnp.array([[4000.0], [4000.0], [4001.0], [4001.0]], f32),
                     (1, _NPAD - _N))
    a4 = jnp.concatenate([anchors.T.astype(f32), pad_a], axis=1)
    aarea = ((a4[2:3] - a4[0:1]) * (a4[3:4] - a4[1:2]))
    aT = jnp.concatenate([a4, aarea], axis=0)
    colval = jnp.concatenate(
        [jnp.ones((1, _N), f32), jnp.zeros((1, _NPAD - _N), f32)], axis=1)
    pad_g = jnp.tile(
        jnp.array([[-4000.0, -4000.0, -3999.0, -3999.0]], f32),
        (_B, _GP - _G, 1))
    gtp = jnp.concatenate([gt_boxes.astype(f32), pad_g], axis=1)
    area = ((gtp[:, :, 2] - gtp[:, :, 0])
            * (gtp[:, :, 3] - gtp[:, :, 1]))[:, :, None]
    rowpen = jnp.concatenate(
        [jnp.zeros((_B, _G, 1), f32), jnp.ones((_B, _GP - _G, 1), f32)],
        axis=1)
    gtc = jnp.concatenate(
        [gtp, area, rowpen, jnp.zeros((_B, _GP, 2), f32)], axis=2)
    gt8 = jnp.concatenate(
        [jnp.where(rowpen.transpose(0, 2, 1) > 0, 0.0,
                   gtp.transpose(0, 2, 1)),
         jnp.zeros((_B, 4, _GP), f32)], axis=1)
    regT = jnp.concatenate(
        [bbox_regression.astype(f32).transpose(0, 2, 1),
         jnp.zeros((_B, 4, _NPAD - _N), f32)], axis=2)

    out = pl.pallas_call(
        _body,
        grid=(_B,),
        in_specs=[
            pl.BlockSpec((5, _NPAD), lambda b: (0, 0)),
            pl.BlockSpec((1, _NPAD), lambda b: (0, 0)),
            pl.BlockSpec((1, _GP, 8), lambda b: (b, 0, 0)),
            pl.BlockSpec((1, 8, _GP), lambda b: (b, 0, 0)),
            pl.BlockSpec((1, 4, _NPAD), lambda b: (b, 0, 0)),
        ],
        out_specs=pl.BlockSpec((1, 1), lambda b: (0, 0)),
        out_shape=jax.ShapeDtypeStruct((1, 1), f32),
        scratch_shapes=[pltpu.VMEM((_GP, _NPAD), f32)],
    )(aT, colval, gtc, gt8, regT)
    return out[0, 0]
